# baseline (speedup 1.0000x reference)
import functools

import jax
import jax.numpy as jnp
from jax.experimental import pallas as pl
from jax.experimental.pallas import tpu as pltpu

_NUM_CODES = 1024
_CODE_DIM = 256
_BETA = 0.25
_BM = 2304
_CHUNK = 256


def _csq_body(cb_ref, csq_ref):
    cb = cb_ref[...]
    csq_ref[...] = jnp.sum(cb * cb, axis=1)[None, :]


def _vq_body(x_ref, cb_ref, csq_ref, qst_ref, idx_ref, loss_ref, sse_ref):
    i = pl.program_id(0)
    n = pl.num_programs(0)
    cb = cb_ref[...]

    @pl.when(i == 0)
    def _init():
        sse_ref[...] = jnp.zeros_like(sse_ref)

    x = x_ref[...]
    zsq = jnp.sum(x * x, axis=1, keepdims=True)
    ciota = jax.lax.broadcasted_iota(jnp.int32, (_BM, _CHUNK), 1)
    m = None
    idx = None
    for k in range(_NUM_CODES // _CHUNK):
        cbk = cb[k * _CHUNK : (k + 1) * _CHUNK, :]
        dotk = jax.lax.dot_general(
            x, cbk, (((1,), (1,)), ((), ())), preferred_element_type=jnp.float32
        )
        dk = (zsq + csq_ref[:, k * _CHUNK : (k + 1) * _CHUNK]) - 2.0 * dotk
        mk = jnp.min(dk, axis=1, keepdims=True)
        tk = jnp.min(jnp.where(dk == mk, ciota + k * _CHUNK, _NUM_CODES), axis=1)
        if m is None:
            m, idx = mk, tk
        else:
            keep = mk < m
            idx = jnp.where(keep[:, 0], tk, idx)
            m = jnp.minimum(m, mk)
    idx_ref[...] = idx[None, None, :]

    iota = jax.lax.broadcasted_iota(jnp.int32, (_BM, _NUM_CODES), 1)
    onehot = (iota == idx[:, None]).astype(jnp.float32)
    q = jax.lax.dot_general(
        onehot, cb, (((1,), (0,)), ((), ())), preferred_element_type=jnp.float32
    )
    qst_ref[...] = q
    sse_ref[...] = sse_ref[...] + jnp.sum(m)

    @pl.when(i == n - 1)
    def _fin():
        mse = sse_ref[0, 0] / (n * _BM * _CODE_DIM)
        loss_ref[...] = (mse + _BETA * mse)[None, None]


def kernel(z, codebook):
    B, T, D = z.shape
    flat = z.reshape(-1, D)
    N = flat.shape[0]
    nblk = N // _BM

    csq = pl.pallas_call(
        _csq_body,
        out_shape=jax.ShapeDtypeStruct((1, _NUM_CODES), jnp.float32),
    )(codebook)

    qst, idx3, loss = pl.pallas_call(
        _vq_body,
        grid=(nblk,),
        in_specs=[
            pl.BlockSpec((_BM, D), lambda i: (i, 0)),
            pl.BlockSpec((_NUM_CODES, D), lambda i: (0, 0)),
            pl.BlockSpec((1, _NUM_CODES), lambda i: (0, 0)),
        ],
        out_specs=[
            pl.BlockSpec((_BM, D), lambda i: (i, 0)),
            pl.BlockSpec((1, 1, _BM), lambda i: (i, 0, 0)),
            pl.BlockSpec((1, 1), lambda i: (0, 0)),
        ],
        out_shape=[
            jax.ShapeDtypeStruct((N, D), jnp.float32),
            jax.ShapeDtypeStruct((nblk, 1, _BM), jnp.int32),
            jax.ShapeDtypeStruct((1, 1), jnp.float32),
        ],
        scratch_shapes=[
            pltpu.VMEM((1, 1), jnp.float32),
        ],
    )(flat, codebook, csq)

    return qst.reshape(B, T, D), loss[0, 0], idx3.reshape(B, T)


# speedup vs baseline: 1.0371x; 1.0371x over previous
import functools

import jax
import jax.numpy as jnp
from jax.experimental import pallas as pl
from jax.experimental.pallas import tpu as pltpu

_NUM_CODES = 1024
_CODE_DIM = 256
_BETA = 0.25
_BM = 2304


def _csq_body(cb_ref, csq_ref):
    cb = cb_ref[...]
    csq_ref[...] = jnp.sum(cb * cb, axis=1)[None, :]


def _vq_body(x_ref, cb_ref, csq_ref, qst_ref, idx_ref, loss_ref, sse_ref):
    i = pl.program_id(0)
    n = pl.num_programs(0)
    cb = cb_ref[...]

    @pl.when(i == 0)
    def _init():
        sse_ref[...] = jnp.zeros_like(sse_ref)

    x = x_ref[...]
    zsq = jnp.sum(x * x, axis=1, keepdims=True)
    dot = jax.lax.dot_general(
        x, cb, (((1,), (1,)), ((), ())), preferred_element_type=jnp.float32
    )
    d = (zsq + csq_ref[...]) - 2.0 * dot
    m = jnp.min(d, axis=1, keepdims=True)
    iota = jax.lax.broadcasted_iota(jnp.int32, (_BM, _NUM_CODES), 1)
    idx = jnp.min(jnp.where(d == m, iota, _NUM_CODES), axis=1)
    idx_ref[...] = idx[:, None]

    onehot = (iota == idx[:, None]).astype(jnp.float32)
    q = jax.lax.dot_general(
        onehot, cb, (((1,), (0,)), ((), ())), preferred_element_type=jnp.float32
    )
    qst_ref[...] = q
    sse_ref[...] = sse_ref[...] + jnp.sum(m)

    @pl.when(i == n - 1)
    def _fin():
        mse = sse_ref[0, 0] / (n * _BM * _CODE_DIM)
        loss_ref[...] = (mse + _BETA * mse)[None, None]


def kernel(z, codebook):
    B, T, D = z.shape
    flat = z.reshape(-1, D)
    N = flat.shape[0]
    nblk = N // _BM

    csq = pl.pallas_call(
        _csq_body,
        out_shape=jax.ShapeDtypeStruct((1, _NUM_CODES), jnp.float32),
    )(codebook)

    qst, idx3, loss = pl.pallas_call(
        _vq_body,
        grid=(nblk,),
        in_specs=[
            pl.BlockSpec((_BM, D), lambda i: (i, 0)),
            pl.BlockSpec((_NUM_CODES, D), lambda i: (0, 0)),
            pl.BlockSpec((1, _NUM_CODES), lambda i: (0, 0)),
        ],
        out_specs=[
            pl.BlockSpec((_BM, D), lambda i: (i, 0)),
            pl.BlockSpec((_BM, 1), lambda i: (i, 0)),
            pl.BlockSpec((1, 1), lambda i: (0, 0)),
        ],
        out_shape=[
            jax.ShapeDtypeStruct((N, D), jnp.float32),
            jax.ShapeDtypeStruct((N, 1), jnp.int32),
            jax.ShapeDtypeStruct((1, 1), jnp.float32),
        ],
        scratch_shapes=[
            pltpu.VMEM((1, 1), jnp.float32),
        ],
    )(flat, codebook, csq)

    return qst.reshape(B, T, D), loss[0, 0], idx3.reshape(B, T)


# speedup vs baseline: 1.0458x; 1.0084x over previous
import functools

import jax
import jax.numpy as jnp
from jax.experimental import pallas as pl
from jax.experimental.pallas import tpu as pltpu

_NUM_CODES = 1024
_CODE_DIM = 256
_BETA = 0.25
_BM = 2304


def _csq_body(cb_ref, csq_ref):
    cb = cb_ref[...]
    csq_ref[...] = jnp.sum(cb * cb, axis=1)[None, :]


def _vq_body(x_ref, cb_ref, csq_ref, qst_ref, idx_ref, loss_ref, sse_ref):
    i = pl.program_id(0)
    n = pl.num_programs(0)
    cb = cb_ref[...]

    @pl.when(i == 0)
    def _init():
        sse_ref[...] = jnp.zeros_like(sse_ref)

    x = x_ref[...]
    zsq = jnp.sum(x * x, axis=1, keepdims=True)
    dot = jax.lax.dot_general(
        x, cb, (((1,), (1,)), ((), ())), preferred_element_type=jnp.float32
    )
    d = (zsq + csq_ref[...]) - 2.0 * dot
    m = jnp.min(d, axis=1, keepdims=True)
    iota = jax.lax.broadcasted_iota(jnp.int32, (_BM, _NUM_CODES), 1)
    idx = jnp.min(jnp.where(d == m, iota, _NUM_CODES), axis=1)
    idx_ref[...] = idx[None, None, :]

    onehot = (iota == idx[:, None]).astype(jnp.float32)
    q = jax.lax.dot_general(
        onehot, cb, (((1,), (0,)), ((), ())), preferred_element_type=jnp.float32
    )
    qst_ref[...] = q
    sse_ref[...] = sse_ref[...] + jnp.sum(m)

    @pl.when(i == n - 1)
    def _fin():
        mse = sse_ref[0, 0] / (n * _BM * _CODE_DIM)
        loss_ref[...] = (mse + _BETA * mse)[None, None]


def kernel(z, codebook):
    B, T, D = z.shape
    flat = z.reshape(-1, D)
    N = flat.shape[0]
    nblk = N // _BM

    csq = pl.pallas_call(
        _csq_body,
        out_shape=jax.ShapeDtypeStruct((1, _NUM_CODES), jnp.float32),
    )(codebook)

    qst, idx3, loss = pl.pallas_call(
        _vq_body,
        grid=(nblk,),
        in_specs=[
            pl.BlockSpec((_BM, D), lambda i: (i, 0)),
            pl.BlockSpec((_NUM_CODES, D), lambda i: (0, 0)),
            pl.BlockSpec((1, _NUM_CODES), lambda i: (0, 0)),
        ],
        out_specs=[
            pl.BlockSpec((_BM, D), lambda i: (i, 0)),
            pl.BlockSpec((1, 1, _BM), lambda i: (i, 0, 0)),
            pl.BlockSpec((1, 1), lambda i: (0, 0)),
        ],
        out_shape=[
            jax.ShapeDtypeStruct((N, D), jnp.float32),
            jax.ShapeDtypeStruct((nblk, 1, _BM), jnp.int32),
            jax.ShapeDtypeStruct((1, 1), jnp.float32),
        ],
        scratch_shapes=[
            pltpu.VMEM((1, 1), jnp.float32),
        ],
    )(flat, codebook, csq)

    return qst.reshape(B, T, D), loss[0, 0], idx3.reshape(B, T)
